# baseline (speedup 1.0000x reference)
import functools

import jax
import jax.numpy as jnp
from jax import lax
from jax.experimental import pallas as pl
from jax.experimental.pallas import tpu as pltpu
from jax.experimental.pallas import tpu_sc as plsc

N = 4096
C = 64
BR = 512
CARBON_INTENSITY = 0.4

NTILES = 32
EPT = N // NTILES
NQ = 5
STRIDE = NQ * C
ACCW = 16 * STRIDE


def _sc_segment_partials(seg_hbm, gen_hbm, cons_hbm, out_hbm,
                         seg_v, gen_v, cons_v, acc):
    wid = lax.axis_index("s") * 2 + lax.axis_index("c")
    base = wid * EPT
    pltpu.sync_copy(seg_hbm.at[pl.ds(base, EPT)], seg_v)
    pltpu.sync_copy(gen_hbm.at[pl.ds(base, EPT)], gen_v)
    pltpu.sync_copy(cons_hbm.at[pl.ds(base, EPT)], cons_v)

    zeros16 = jnp.zeros((16,), jnp.float32)
    neg16 = jnp.full((16,), -jnp.inf, jnp.float32)
    ones16 = jnp.ones((16,), jnp.float32)
    for lane in range(16):
        off = lane * STRIDE
        for k in range(4 * C // 16):
            acc[pl.ds(off + k * 16, 16)] = zeros16
        for k in range(C // 16):
            acc[pl.ds(off + 4 * C + k * 16, 16)] = neg16

    lanes = lax.iota(jnp.int32, 16)
    for j in range(EPT // 16):
        sl = pl.ds(j * 16, 16)
        seg16 = seg_v[sl]
        g16 = gen_v[sl]
        c16 = cons_v[sl]
        idx0 = lanes * STRIDE + seg16
        c0 = plsc.load_gather(acc, [idx0])
        plsc.store_scatter(acc, [idx0], c0 + ones16)
        c1 = plsc.load_gather(acc, [idx0 + C])
        plsc.store_scatter(acc, [idx0 + C], c1 + g16)
        c2 = plsc.load_gather(acc, [idx0 + 2 * C])
        plsc.store_scatter(acc, [idx0 + 2 * C], c2 + c16)
        c3 = plsc.load_gather(acc, [idx0 + 3 * C])
        plsc.store_scatter(acc, [idx0 + 3 * C], c3 + c16 * c16)
        idx4 = idx0 + 4 * C
        cur = plsc.load_gather(acc, [idx4])
        plsc.store_scatter(acc, [idx4], jnp.maximum(cur, c16))

    pltpu.sync_copy(acc, out_hbm.at[wid])


_sc_mesh = plsc.VectorSubcoreMesh(core_axis_name="c", subcore_axis_name="s")
_sc_partials = functools.partial(
    pl.kernel, mesh=_sc_mesh,
    out_type=jax.ShapeDtypeStruct((NTILES, ACCW), jnp.float32),
    compiler_params=pltpu.CompilerParams(needs_layout_passes=False),
    scratch_types=[
        pltpu.VMEM((EPT,), jnp.int32),
        pltpu.VMEM((EPT,), jnp.float32),
        pltpu.VMEM((EPT,), jnp.float32),
        pltpu.VMEM((ACCW,), jnp.float32),
    ],
)(_sc_segment_partials)


def _stream_body(seg_rows_ref, seg_full_ref, s_ref, e_ref,
                 colsum_ref, sent_ref, del_ref, colacc):
    i = pl.program_id(0)
    s = s_ref[0]
    e = e_ref[0]
    p = s * e
    psum = jnp.sum(p, axis=0)

    @pl.when(i == 0)
    def _():
        colacc[0, :] = psum

    @pl.when(i != 0)
    def _():
        colacc[0, :] = colacc[0, :] + psum

    seg_rows = seg_rows_ref[0]
    seg_full = seg_full_ref[0]
    mask = seg_rows[:, None] == seg_full[None, :]
    sent_ref[0, :] = jnp.sum(jnp.where(mask, s, 0.0), axis=1)
    del_ref[0, :] = jnp.sum(jnp.where(mask, p, 0.0), axis=1)

    @pl.when(i == (N // BR) - 1)
    def _():
        colsum_ref[0, :] = colacc[0, :]


def _combine_body(seg_ref, cons_ref, w_ref, colsum_ref, sent_ref, del_ref,
                  cnt_ref, tg_ref, tc_ref, sq_ref, mx_ref, out_ref):
    seg = seg_ref[0]
    cons = cons_ref[0]
    colsum = colsum_ref[0]
    sent = sent_ref[0]
    dele = del_ref[0]

    count = jnp.sum(cnt_ref[...], axis=0)
    total_gen = jnp.sum(tg_ref[...], axis=0)
    total_cons = jnp.sum(tc_ref[...], axis=0)
    sum_sq = jnp.sum(sq_ref[...], axis=0)
    peak_without = jnp.max(mx_ref[...], axis=0)

    cl = jax.lax.broadcasted_iota(jnp.int32, (C, N), 0)
    m = seg[None, :] == cl
    zeros = jnp.zeros((C, N), jnp.float32)
    neg = jnp.full((C, N), -jnp.inf, jnp.float32)

    net = cons - colsum
    peak_with = jnp.max(jnp.where(m, net[None, :], neg), axis=1)
    total_sent = jnp.sum(jnp.where(m, sent[None, :], zeros), axis=1)
    total_del = jnp.sum(jnp.where(m, dele[None, :], zeros), axis=1)

    local_energy_used = jnp.minimum(total_gen, total_cons)
    ssr = local_energy_used / (total_cons + 1e-06)
    peak_reduction = (peak_without - peak_with) / (peak_without + 1e-06)

    mean_c = total_cons / jnp.maximum(count, 1.0)
    var = (sum_sq - count * mean_c * mean_c) / jnp.maximum(count - 1.0, 1.0)
    std = jnp.sqrt(jnp.maximum(var, 1e-12))
    diversity_index = std / (mean_c + 1e-06)

    safe_sent = jnp.where(total_sent > 0, total_sent, 1.0)
    sharing_efficiency = jnp.where(total_sent > 0, total_del / safe_sent, 1.0)

    carbon_saved = local_energy_used * CARBON_INTENSITY

    w = w_ref[0]
    overall = (w[0] * ssr + w[1] * peak_reduction + w[2] * diversity_index +
               w[3] * sharing_efficiency + w[4] * (carbon_saved / 100.0))

    out_ref[...] = jnp.stack(
        [ssr, peak_reduction, diversity_index, sharing_efficiency,
         carbon_saved, overall], axis=1)


@jax.jit
def kernel(cluster_assignments, generation, consumption, sharing_matrix,
           efficiency_matrix, weights):
    seg = cluster_assignments.astype(jnp.int32)
    cons = consumption
    w = weights.reshape(1, -1)

    grid = N // BR
    colsum, sent_row, del_row = pl.pallas_call(
        _stream_body,
        grid=(grid,),
        in_specs=[
            pl.BlockSpec((1, BR), lambda i: (0, i)),
            pl.BlockSpec((1, N), lambda i: (0, 0)),
            pl.BlockSpec((1, BR, N), lambda i: (0, i, 0)),
            pl.BlockSpec((1, BR, N), lambda i: (0, i, 0)),
        ],
        out_specs=[
            pl.BlockSpec((1, N), lambda i: (0, 0)),
            pl.BlockSpec((1, BR), lambda i: (0, i)),
            pl.BlockSpec((1, BR), lambda i: (0, i)),
        ],
        out_shape=[
            jax.ShapeDtypeStruct((1, N), jnp.float32),
            jax.ShapeDtypeStruct((1, N), jnp.float32),
            jax.ShapeDtypeStruct((1, N), jnp.float32),
        ],
        scratch_shapes=[pltpu.VMEM((1, N), jnp.float32)],
    )(seg, seg, sharing_matrix, efficiency_matrix)

    sc_out = _sc_partials(seg.reshape(N), generation.reshape(N),
                          cons.reshape(N))
    banks = sc_out.reshape(NTILES * 16, NQ, C)
    cnt_p = banks[:, 0, :]
    tg_p = banks[:, 1, :]
    tc_p = banks[:, 2, :]
    sq_p = banks[:, 3, :]
    mx_p = banks[:, 4, :]

    out = pl.pallas_call(
        _combine_body,
        in_specs=[pl.BlockSpec(x.shape, lambda: (0,) * x.ndim)
                  for x in (seg, cons, w, colsum, sent_row, del_row,
                            cnt_p, tg_p, tc_p, sq_p, mx_p)],
        out_specs=pl.BlockSpec((C, 6), lambda: (0, 0)),
        out_shape=jax.ShapeDtypeStruct((C, 6), jnp.float32),
    )(seg, cons, w, colsum, sent_row, del_row,
      cnt_p, tg_p, tc_p, sq_p, mx_p)

    return out


# speedup vs baseline: 1.0261x; 1.0261x over previous
import functools

import jax
import jax.numpy as jnp
from jax import lax
from jax.experimental import pallas as pl
from jax.experimental.pallas import tpu as pltpu
from jax.experimental.pallas import tpu_sc as plsc

N = 4096
C = 64
BR = 512
CARBON_INTENSITY = 0.4

NTILES = 32
EPT = N // NTILES
NQ = 5
STRIDE = NQ * C
ACCW = 16 * STRIDE


def _sc_segment_partials(seg_hbm, gen_hbm, cons_hbm, out_hbm,
                         seg_v, gen_v, cons_v, acc):
    wid = lax.axis_index("s") * 2 + lax.axis_index("c")
    base = wid * EPT
    pltpu.sync_copy(seg_hbm.at[pl.ds(base, EPT)], seg_v)
    pltpu.sync_copy(gen_hbm.at[pl.ds(base, EPT)], gen_v)
    pltpu.sync_copy(cons_hbm.at[pl.ds(base, EPT)], cons_v)

    zeros16 = jnp.zeros((16,), jnp.float32)
    neg16 = jnp.full((16,), -jnp.inf, jnp.float32)
    ones16 = jnp.ones((16,), jnp.float32)
    for lane in range(16):
        off = lane * STRIDE
        for k in range(4 * C // 16):
            acc[pl.ds(off + k * 16, 16)] = zeros16
        for k in range(C // 16):
            acc[pl.ds(off + 4 * C + k * 16, 16)] = neg16

    lanes = lax.iota(jnp.int32, 16)
    for j in range(EPT // 16):
        sl = pl.ds(j * 16, 16)
        seg16 = seg_v[sl]
        g16 = gen_v[sl]
        c16 = cons_v[sl]
        idx0 = lanes * STRIDE + seg16
        c0 = plsc.load_gather(acc, [idx0])
        plsc.store_scatter(acc, [idx0], c0 + ones16)
        c1 = plsc.load_gather(acc, [idx0 + C])
        plsc.store_scatter(acc, [idx0 + C], c1 + g16)
        c2 = plsc.load_gather(acc, [idx0 + 2 * C])
        plsc.store_scatter(acc, [idx0 + 2 * C], c2 + c16)
        c3 = plsc.load_gather(acc, [idx0 + 3 * C])
        plsc.store_scatter(acc, [idx0 + 3 * C], c3 + c16 * c16)
        idx4 = idx0 + 4 * C
        cur = plsc.load_gather(acc, [idx4])
        plsc.store_scatter(acc, [idx4], jnp.maximum(cur, c16))

    pltpu.sync_copy(acc, out_hbm.at[wid])


_sc_mesh = plsc.VectorSubcoreMesh(core_axis_name="c", subcore_axis_name="s")
_sc_partials = functools.partial(
    pl.kernel, mesh=_sc_mesh,
    out_type=jax.ShapeDtypeStruct((NTILES, ACCW), jnp.float32),
    compiler_params=pltpu.CompilerParams(needs_layout_passes=False),
    scratch_types=[
        pltpu.VMEM((EPT,), jnp.int32),
        pltpu.VMEM((EPT,), jnp.float32),
        pltpu.VMEM((EPT,), jnp.float32),
        pltpu.VMEM((ACCW,), jnp.float32),
    ],
)(_sc_segment_partials)


def _stream_body(seg_rows_ref, seg_full_ref, s_ref, e_ref,
                 colsum_ref, sent_ref, del_ref, colacc):
    i = pl.program_id(0)
    s = s_ref[0]
    e = e_ref[0]
    p = s * e
    psum = jnp.sum(p, axis=0)

    @pl.when(i == 0)
    def _():
        colacc[0, :] = psum

    @pl.when(i != 0)
    def _():
        colacc[0, :] = colacc[0, :] + psum

    seg_rows = seg_rows_ref[0]
    seg_full = seg_full_ref[0]
    mask = seg_rows[:, None] == seg_full[None, :]
    sent_ref[0, :] = jnp.sum(jnp.where(mask, s, 0.0), axis=1)
    del_ref[0, :] = jnp.sum(jnp.where(mask, p, 0.0), axis=1)

    @pl.when(i == (N // BR) - 1)
    def _():
        colsum_ref[0, :] = colacc[0, :]


def _combine_body(seg_ref, cons_ref, w_ref, colsum_ref, sent_ref, del_ref,
                  banks_ref, out_ref):
    seg = seg_ref[0]
    cons = cons_ref[0]
    colsum = colsum_ref[0]
    sent = sent_ref[0]
    dele = del_ref[0]

    banks = banks_ref[...]
    count = jnp.sum(banks[:, 0:C], axis=0)
    total_gen = jnp.sum(banks[:, C:2 * C], axis=0)
    total_cons = jnp.sum(banks[:, 2 * C:3 * C], axis=0)
    sum_sq = jnp.sum(banks[:, 3 * C:4 * C], axis=0)
    peak_without = jnp.max(banks[:, 4 * C:5 * C], axis=0)

    cl = jax.lax.broadcasted_iota(jnp.int32, (C, N), 0)
    m = seg[None, :] == cl
    zeros = jnp.zeros((C, N), jnp.float32)
    neg = jnp.full((C, N), -jnp.inf, jnp.float32)

    net = cons - colsum
    peak_with = jnp.max(jnp.where(m, net[None, :], neg), axis=1)
    total_sent = jnp.sum(jnp.where(m, sent[None, :], zeros), axis=1)
    total_del = jnp.sum(jnp.where(m, dele[None, :], zeros), axis=1)

    local_energy_used = jnp.minimum(total_gen, total_cons)
    ssr = local_energy_used / (total_cons + 1e-06)
    peak_reduction = (peak_without - peak_with) / (peak_without + 1e-06)

    mean_c = total_cons / jnp.maximum(count, 1.0)
    var = (sum_sq - count * mean_c * mean_c) / jnp.maximum(count - 1.0, 1.0)
    std = jnp.sqrt(jnp.maximum(var, 1e-12))
    diversity_index = std / (mean_c + 1e-06)

    safe_sent = jnp.where(total_sent > 0, total_sent, 1.0)
    sharing_efficiency = jnp.where(total_sent > 0, total_del / safe_sent, 1.0)

    carbon_saved = local_energy_used * CARBON_INTENSITY

    w = w_ref[0]
    overall = (w[0] * ssr + w[1] * peak_reduction + w[2] * diversity_index +
               w[3] * sharing_efficiency + w[4] * (carbon_saved / 100.0))

    out_ref[...] = jnp.stack(
        [ssr, peak_reduction, diversity_index, sharing_efficiency,
         carbon_saved, overall], axis=1)


@jax.jit
def kernel(cluster_assignments, generation, consumption, sharing_matrix,
           efficiency_matrix, weights):
    seg = cluster_assignments.astype(jnp.int32)
    cons = consumption
    w = weights.reshape(1, -1)

    grid = N // BR
    colsum, sent_row, del_row = pl.pallas_call(
        _stream_body,
        grid=(grid,),
        in_specs=[
            pl.BlockSpec((1, BR), lambda i: (0, i)),
            pl.BlockSpec((1, N), lambda i: (0, 0)),
            pl.BlockSpec((1, BR, N), lambda i: (0, i, 0)),
            pl.BlockSpec((1, BR, N), lambda i: (0, i, 0)),
        ],
        out_specs=[
            pl.BlockSpec((1, N), lambda i: (0, 0)),
            pl.BlockSpec((1, BR), lambda i: (0, i)),
            pl.BlockSpec((1, BR), lambda i: (0, i)),
        ],
        out_shape=[
            jax.ShapeDtypeStruct((1, N), jnp.float32),
            jax.ShapeDtypeStruct((1, N), jnp.float32),
            jax.ShapeDtypeStruct((1, N), jnp.float32),
        ],
        scratch_shapes=[pltpu.VMEM((1, N), jnp.float32)],
    )(seg, seg, sharing_matrix, efficiency_matrix)

    sc_out = _sc_partials(seg.reshape(N), generation.reshape(N),
                          cons.reshape(N))
    banks = sc_out.reshape(NTILES * 16, NQ * C)

    out = pl.pallas_call(
        _combine_body,
        in_specs=[pl.BlockSpec(x.shape, lambda: (0,) * x.ndim)
                  for x in (seg, cons, w, colsum, sent_row, del_row, banks)],
        out_specs=pl.BlockSpec((C, 6), lambda: (0, 0)),
        out_shape=jax.ShapeDtypeStruct((C, 6), jnp.float32),
    )(seg, cons, w, colsum, sent_row, del_row, banks)

    return out


# speedup vs baseline: 1.0420x; 1.0155x over previous
import functools

import jax
import jax.numpy as jnp
from jax import lax
from jax.experimental import pallas as pl
from jax.experimental.pallas import tpu as pltpu
from jax.experimental.pallas import tpu_sc as plsc

N = 4096
C = 64
BR = 512
CARBON_INTENSITY = 0.4

NTILES = 16
EPT = N // NTILES
NQ = 5
STRIDE = NQ * C
ACCW = 16 * STRIDE


def _sc_segment_partials(init_hbm, seg_hbm, gen_hbm, cons_hbm, out_hbm,
                         seg_v, gen_v, cons_v, acc):
    wid = lax.axis_index("s")
    base = wid * EPT
    pltpu.sync_copy(seg_hbm.at[pl.ds(base, EPT)], seg_v)
    pltpu.sync_copy(gen_hbm.at[pl.ds(base, EPT)], gen_v)
    pltpu.sync_copy(cons_hbm.at[pl.ds(base, EPT)], cons_v)
    pltpu.sync_copy(init_hbm, acc)

    ones16 = jnp.ones((16,), jnp.float32)
    lanes = lax.iota(jnp.int32, 16)
    for j in range(EPT // 16):
        sl = pl.ds(j * 16, 16)
        seg16 = seg_v[sl]
        g16 = gen_v[sl]
        c16 = cons_v[sl]
        idx0 = lanes * STRIDE + seg16
        c0 = plsc.load_gather(acc, [idx0])
        plsc.store_scatter(acc, [idx0], c0 + ones16)
        c1 = plsc.load_gather(acc, [idx0 + C])
        plsc.store_scatter(acc, [idx0 + C], c1 + g16)
        c2 = plsc.load_gather(acc, [idx0 + 2 * C])
        plsc.store_scatter(acc, [idx0 + 2 * C], c2 + c16)
        c3 = plsc.load_gather(acc, [idx0 + 3 * C])
        plsc.store_scatter(acc, [idx0 + 3 * C], c3 + c16 * c16)
        idx4 = idx0 + 4 * C
        cur = plsc.load_gather(acc, [idx4])
        plsc.store_scatter(acc, [idx4], jnp.maximum(cur, c16))

    pltpu.sync_copy(acc, out_hbm.at[wid])


_sc_mesh = plsc.VectorSubcoreMesh(core_axis_name="c", subcore_axis_name="s",
                                  num_cores=1)
_sc_partials = functools.partial(
    pl.kernel, mesh=_sc_mesh,
    out_type=jax.ShapeDtypeStruct((NTILES, ACCW), jnp.float32),
    compiler_params=pltpu.CompilerParams(needs_layout_passes=False),
    scratch_types=[
        pltpu.VMEM((EPT,), jnp.int32),
        pltpu.VMEM((EPT,), jnp.float32),
        pltpu.VMEM((EPT,), jnp.float32),
        pltpu.VMEM((ACCW,), jnp.float32),
    ],
)(_sc_segment_partials)


def _stream_body(seg_rows_ref, seg_full_ref, s_ref, e_ref,
                 colsum_ref, sent_ref, del_ref, colacc):
    i = pl.program_id(0)
    s = s_ref[0]
    e = e_ref[0]
    p = s * e
    psum = jnp.sum(p, axis=0)

    @pl.when(i == 0)
    def _():
        colacc[0, :] = psum

    @pl.when(i != 0)
    def _():
        colacc[0, :] = colacc[0, :] + psum

    seg_rows = seg_rows_ref[0]
    seg_full = seg_full_ref[0]
    mask = seg_rows[:, None] == seg_full[None, :]
    sent_ref[0, :] = jnp.sum(jnp.where(mask, s, 0.0), axis=1)
    del_ref[0, :] = jnp.sum(jnp.where(mask, p, 0.0), axis=1)

    @pl.when(i == (N // BR) - 1)
    def _():
        colsum_ref[0, :] = colacc[0, :]


def _combine_body(seg_ref, cons_ref, w_ref, colsum_ref, sent_ref, del_ref,
                  banks_ref, out_ref):
    seg = seg_ref[0]
    cons = cons_ref[0]
    colsum = colsum_ref[0]
    sent = sent_ref[0]
    dele = del_ref[0]

    banks = banks_ref[...]
    count = jnp.sum(banks[:, 0:C], axis=0)
    total_gen = jnp.sum(banks[:, C:2 * C], axis=0)
    total_cons = jnp.sum(banks[:, 2 * C:3 * C], axis=0)
    sum_sq = jnp.sum(banks[:, 3 * C:4 * C], axis=0)
    peak_without = jnp.max(banks[:, 4 * C:5 * C], axis=0)

    cl = jax.lax.broadcasted_iota(jnp.int32, (C, N), 0)
    m = seg[None, :] == cl
    zeros = jnp.zeros((C, N), jnp.float32)
    neg = jnp.full((C, N), -jnp.inf, jnp.float32)

    net = cons - colsum
    peak_with = jnp.max(jnp.where(m, net[None, :], neg), axis=1)
    total_sent = jnp.sum(jnp.where(m, sent[None, :], zeros), axis=1)
    total_del = jnp.sum(jnp.where(m, dele[None, :], zeros), axis=1)

    local_energy_used = jnp.minimum(total_gen, total_cons)
    ssr = local_energy_used / (total_cons + 1e-06)
    peak_reduction = (peak_without - peak_with) / (peak_without + 1e-06)

    mean_c = total_cons / jnp.maximum(count, 1.0)
    var = (sum_sq - count * mean_c * mean_c) / jnp.maximum(count - 1.0, 1.0)
    std = jnp.sqrt(jnp.maximum(var, 1e-12))
    diversity_index = std / (mean_c + 1e-06)

    safe_sent = jnp.where(total_sent > 0, total_sent, 1.0)
    sharing_efficiency = jnp.where(total_sent > 0, total_del / safe_sent, 1.0)

    carbon_saved = local_energy_used * CARBON_INTENSITY

    w = w_ref[0]
    overall = (w[0] * ssr + w[1] * peak_reduction + w[2] * diversity_index +
               w[3] * sharing_efficiency + w[4] * (carbon_saved / 100.0))

    out_ref[...] = jnp.stack(
        [ssr, peak_reduction, diversity_index, sharing_efficiency,
         carbon_saved, overall], axis=1)


@jax.jit
def kernel(cluster_assignments, generation, consumption, sharing_matrix,
           efficiency_matrix, weights):
    seg = cluster_assignments.astype(jnp.int32)
    cons = consumption
    w = weights.reshape(1, -1)

    grid = N // BR
    colsum, sent_row, del_row = pl.pallas_call(
        _stream_body,
        grid=(grid,),
        in_specs=[
            pl.BlockSpec((1, BR), lambda i: (0, i)),
            pl.BlockSpec((1, N), lambda i: (0, 0)),
            pl.BlockSpec((1, BR, N), lambda i: (0, i, 0)),
            pl.BlockSpec((1, BR, N), lambda i: (0, i, 0)),
        ],
        out_specs=[
            pl.BlockSpec((1, N), lambda i: (0, 0)),
            pl.BlockSpec((1, BR), lambda i: (0, i)),
            pl.BlockSpec((1, BR), lambda i: (0, i)),
        ],
        out_shape=[
            jax.ShapeDtypeStruct((1, N), jnp.float32),
            jax.ShapeDtypeStruct((1, N), jnp.float32),
            jax.ShapeDtypeStruct((1, N), jnp.float32),
        ],
        scratch_shapes=[pltpu.VMEM((1, N), jnp.float32)],
    )(seg, seg, sharing_matrix, efficiency_matrix)

    init = jnp.concatenate(
        [jnp.zeros((4 * C,), jnp.float32),
         jnp.full((C,), -jnp.inf, jnp.float32)])
    init = jnp.tile(init, 16)
    sc_out = _sc_partials(init, seg.reshape(N), generation.reshape(N),
                          cons.reshape(N))
    banks = sc_out.reshape(NTILES * 16, NQ * C)

    out = pl.pallas_call(
        _combine_body,
        in_specs=[pl.BlockSpec(x.shape, lambda: (0,) * x.ndim)
                  for x in (seg, cons, w, colsum, sent_row, del_row, banks)],
        out_specs=pl.BlockSpec((C, 6), lambda: (0, 0)),
        out_shape=jax.ShapeDtypeStruct((C, 6), jnp.float32),
    )(seg, cons, w, colsum, sent_row, del_row, banks)

    return out
